# Initial kernel scaffold; baseline (speedup 1.0000x reference)
import jax
import jax.numpy as jnp
from jax.experimental import pallas as pl


def kernel(query_embeddings, reference_embeddings):
    raise NotImplementedError("write your pallas kernel here")


# speedup vs baseline: 1.1127x; 1.1127x over previous
import jax
import jax.numpy as jnp
from jax import lax
from jax.experimental import pallas as pl
from jax.experimental.pallas import tpu as pltpu

_BQ = 256
_BR = 4096


def _cos_tile(q_ref, r_ref):
    q = q_ref[...]
    r = r_ref[...]
    qn = q * lax.rsqrt(jnp.sum(q * q, axis=1, keepdims=True))
    rn = r * lax.rsqrt(jnp.sum(r * r, axis=1, keepdims=True))
    return lax.dot_general(qn, rn, (((1,), (1,)), ((), ())),
                           preferred_element_type=jnp.float32)


def _minmax_kernel(q_ref, r_ref, mm_ref):
    t = _cos_tile(q_ref, r_ref)
    tmin = jnp.min(t)
    tmax = jnp.max(t)
    first = jnp.logical_and(pl.program_id(0) == 0, pl.program_id(1) == 0)

    @pl.when(first)
    def _init():
        mm_ref[0] = tmin
        mm_ref[1] = tmax

    @pl.when(jnp.logical_not(first))
    def _acc():
        mm_ref[0] = jnp.minimum(mm_ref[0], tmin)
        mm_ref[1] = jnp.maximum(mm_ref[1], tmax)


def _norm_kernel(mm_ref, q_ref, r_ref, o_ref):
    t = _cos_tile(q_ref, r_ref)
    mn = mm_ref[0]
    scale = 1.0 / (mm_ref[1] - mn)
    o_ref[...] = jnp.nan_to_num((t - mn) * scale, nan=0.0)


def kernel(query_embeddings, reference_embeddings):
    q_rows, d = query_embeddings.shape
    k_rows, _ = reference_embeddings.shape
    grid = (k_rows // _BR, q_rows // _BQ)

    minmax = pl.pallas_call(
        _minmax_kernel,
        grid=grid,
        in_specs=[
            pl.BlockSpec((_BQ, d), lambda j, i: (i, 0)),
            pl.BlockSpec((_BR, d), lambda j, i: (j, 0)),
        ],
        out_specs=pl.BlockSpec(memory_space=pltpu.SMEM),
        out_shape=jax.ShapeDtypeStruct((2,), jnp.float32),
    )(query_embeddings, reference_embeddings)

    return pl.pallas_call(
        _norm_kernel,
        grid=grid,
        in_specs=[
            pl.BlockSpec(memory_space=pltpu.SMEM),
            pl.BlockSpec((_BQ, d), lambda j, i: (i, 0)),
            pl.BlockSpec((_BR, d), lambda j, i: (j, 0)),
        ],
        out_specs=pl.BlockSpec((_BQ, _BR), lambda j, i: (i, j)),
        out_shape=jax.ShapeDtypeStruct((q_rows, k_rows), jnp.float32),
    )(minmax, query_embeddings, reference_embeddings)
